# baseline (speedup 1.0000x reference)
import functools

import jax
import jax.numpy as jnp
from jax import lax
from jax.experimental import pallas as pl
from jax.experimental.pallas import tpu as pltpu
from jax.experimental.pallas import tpu_sc as plsc

CTX = 200
EMBED = 128
K = CTX * EMBED
HID = 128
NTRANS = 100000

PAD_B = 256
BK = 2560
KBN = K // BK
BN = 12544
NBN = -(-NTRANS // BN)


def _sc_gather(emb, idx):
    info = plsc.get_sparse_core_info()
    nw = info.num_cores * info.num_subcores
    bpw = PAD_B // nw
    d = emb.shape[1]
    mesh = plsc.VectorSubcoreMesh(core_axis_name="c", subcore_axis_name="s")

    @functools.partial(
        pl.kernel,
        mesh=mesh,
        out_type=jax.ShapeDtypeStruct((PAD_B, d), jnp.float32),
        scratch_types=[
            pltpu.VMEM((bpw,), jnp.int32),
            pltpu.VMEM((bpw, d), jnp.float32),
            pltpu.SemaphoreType.DMA,
        ],
    )
    def gather_k(emb_hbm, idx_hbm, out_hbm, idx_v, rows_v, sem):
        wid = lax.axis_index("s") * info.num_cores + lax.axis_index("c")
        base = wid * bpw
        pltpu.sync_copy(idx_hbm.at[pl.ds(base, bpw)], idx_v)
        pltpu.async_copy(emb_hbm.at[idx_v], rows_v, sem).wait()
        pltpu.sync_copy(rows_v, out_hbm.at[pl.ds(base, bpw)])

    return gather_k(emb, idx)


def _mlp_body(e_ref, w1_ref, b1_ref, w2_ref, b2_ref, out_ref, lse_ref,
              h_ref, m_ref, s_ref):
    i = pl.program_id(0)

    @pl.when(i == 0)
    def _init():
        h_ref[...] = jnp.zeros_like(h_ref)

    @pl.when(i < KBN)
    def _phase_a():
        h_ref[...] += jnp.dot(e_ref[...], w1_ref[...],
                              preferred_element_type=jnp.float32)

    @pl.when(i == KBN - 1)
    def _finish_h():
        h_ref[...] = jnp.maximum(h_ref[...] + b1_ref[...], 0.0)

    @pl.when(i >= KBN)
    def _phase_b():
        j = i - KBN
        o = jnp.dot(h_ref[...], w2_ref[...],
                    preferred_element_type=jnp.float32) + b2_ref[...]
        col = j * BN + lax.broadcasted_iota(jnp.int32, (1, BN), 1)
        valid = col < NTRANS
        o = jnp.where(valid, o, -jnp.inf)
        out_ref[...] = o
        bm = jnp.max(o)
        bs = jnp.sum(jnp.where(valid, jnp.exp(o - bm), 0.0))
        bm_v = jnp.full((1, HID), bm, jnp.float32)
        bs_v = jnp.full((1, HID), bs, jnp.float32)

        @pl.when(j == 0)
        def _first():
            m_ref[...] = bm_v
            s_ref[...] = bs_v

        @pl.when(j > 0)
        def _combine():
            m_old = m_ref[...]
            m_new = jnp.maximum(m_old, bm_v)
            s_ref[...] = (s_ref[...] * jnp.exp(m_old - m_new)
                          + bs_v * jnp.exp(bm_v - m_new))
            m_ref[...] = m_new

        @pl.when(i == KBN + NBN - 1)
        def _final():
            lse_ref[...] = m_ref[...] + jnp.log(s_ref[...])


def _mlp_lse(embeds, w1, b1, w2, b2):
    return pl.pallas_call(
        _mlp_body,
        grid=(KBN + NBN,),
        in_specs=[
            pl.BlockSpec((1, BK), lambda i: (0, jnp.minimum(i, KBN - 1))),
            pl.BlockSpec((BK, HID), lambda i: (jnp.minimum(i, KBN - 1), 0)),
            pl.BlockSpec((1, HID), lambda i: (0, 0)),
            pl.BlockSpec((HID, BN), lambda i: (0, jnp.maximum(i - KBN, 0))),
            pl.BlockSpec((1, BN), lambda i: (0, jnp.maximum(i - KBN, 0))),
        ],
        out_specs=[
            pl.BlockSpec((1, BN), lambda i: (0, jnp.maximum(i - KBN, 0))),
            pl.BlockSpec((1, HID), lambda i: (0, 0)),
        ],
        out_shape=[
            jax.ShapeDtypeStruct((1, NTRANS), jnp.float32),
            jax.ShapeDtypeStruct((1, HID), jnp.float32),
        ],
        scratch_shapes=[
            pltpu.VMEM((1, HID), jnp.float32),
            pltpu.VMEM((1, HID), jnp.float32),
            pltpu.VMEM((1, HID), jnp.float32),
        ],
    )(embeds, w1, b1, w2, b2)


def _norm_body(raw_ref, lse_ref, out_ref):
    out_ref[...] = raw_ref[...] - lse_ref[0, 0]


def _normalize(raw, lse):
    return pl.pallas_call(
        _norm_body,
        grid=(NBN,),
        in_specs=[
            pl.BlockSpec((1, BN), lambda i: (0, i)),
            pl.BlockSpec((1, HID), lambda i: (0, 0)),
        ],
        out_specs=pl.BlockSpec((1, BN), lambda i: (0, i)),
        out_shape=jax.ShapeDtypeStruct((1, NTRANS), jnp.float32),
    )(raw, lse)


def kernel(x, emb, W1, b1, W2, b2):
    idx = jnp.zeros((PAD_B,), jnp.int32).at[:CTX].set(x.astype(jnp.int32))
    rows = _sc_gather(emb, idx)
    embeds = rows[:CTX].reshape(1, K)
    raw, lse = _mlp_lse(embeds, W1, b1.reshape(1, HID), W2,
                        b2.reshape(1, NTRANS))
    return _normalize(raw, lse)
